# baseline (speedup 1.0000x reference)
import functools

import jax
import jax.numpy as jnp
from jax import lax
from jax.experimental import pallas as pl
from jax.experimental.pallas import tpu as pltpu
from jax.experimental.pallas import tpu_sc as plsc

BATCH = 16384
HIST = 50
DIM = 32
NUM_EMB = 1000000

SPLIT = 1 << 18
NSPLIT = 4
VROWS = SPLIT * NSPLIT
LAST_ALIGNED = NUM_EMB - (NUM_EMB % 128)
LAST_FULL = LAST_ALIGNED - 128

_info = plsc.get_sparse_core_info()
NC, NS = _info.num_cores, _info.num_subcores
NW = NC * NS

GROUPS = SPLIT // 128
GROUPS_PER_W = GROUPS // NW


def _relayout_kernel(tt_hbm, out_hbm, in_v, in64_v, out_v, sem):
    wid = lax.axis_index("s") * NC + lax.axis_index("c")
    iota = lax.iota(jnp.int32, 16)

    def group_body(k, carry):
        g = wid * GROUPS_PER_W + k
        q0 = g * 128

        copies = []
        for s in range(NSPLIT):
            c0 = jnp.minimum(s * SPLIT + q0, LAST_FULL)
            c0 = pl.multiple_of(c0, 128)
            copies.append(
                pltpu.async_copy(
                    tt_hbm.at[:, pl.ds(c0, 128)], in_v.at[s], sem
                )
            )
        is_boundary = (3 * SPLIT + q0) == LAST_ALIGNED

        @pl.when(is_boundary)
        def _():
            pltpu.async_copy(
                tt_hbm.at[:, pl.ds(LAST_ALIGNED, NUM_EMB - LAST_ALIGNED)],
                in64_v,
                sem,
            ).wait()

        for cp in copies:
            cp.wait()

        def col_body(c, carry2):
            cvec = jnp.full((16,), c, jnp.int32)
            base = c * 128
            for s in range(NSPLIT):
                for h in range(2):
                    v = plsc.load_gather(in_v, [
                        jnp.full((16,), s, jnp.int32),
                        iota + (16 * h),
                        cvec,
                    ])
                    out_v[pl.ds(base + 32 * s + 16 * h, 16)] = v
            return carry2

        lax.fori_loop(0, 128, col_body, 0, unroll=False)

        @pl.when(is_boundary)
        def _():
            def fix_body(c, carry2):
                cvec = jnp.full((16,), c, jnp.int32)
                for h in range(2):
                    v = plsc.load_gather(in64_v, [iota + (16 * h), cvec])
                    out_v[pl.ds(c * 128 + 96 + 16 * h, 16)] = v
                return carry2

            lax.fori_loop(0, NUM_EMB - LAST_ALIGNED, fix_body, 0,
                          unroll=False)

        pltpu.sync_copy(out_v, out_hbm.at[pl.ds(q0 * 128, 16384)])
        return carry

    lax.fori_loop(0, GROUPS_PER_W, group_body, 0, unroll=False)


def _table_to_split_rows(table):
    table_t = jnp.swapaxes(table, 0, 1)
    mesh = plsc.VectorSubcoreMesh(core_axis_name="c", subcore_axis_name="s")
    run = functools.partial(
        pl.kernel,
        mesh=mesh,
        out_type=jax.ShapeDtypeStruct((VROWS * DIM,), jnp.float32),
        scratch_types=[
            pltpu.VMEM((NSPLIT, DIM, 128), jnp.float32),
            pltpu.VMEM((DIM, NUM_EMB - LAST_ALIGNED), jnp.float32),
            pltpu.VMEM((16384,), jnp.float32),
            pltpu.SemaphoreType.DMA,
        ],
        compiler_params=pltpu.CompilerParams(
            use_tc_tiling_on_sc=True, needs_layout_passes=False
        ),
    )(_relayout_kernel)
    return run(table_t).reshape(VROWS, DIM)


BAGS_PER_W = BATCH // NW
CHUNK_BAGS = 32
CHUNK_IDX = CHUNK_BAGS * HIST
N_CHUNKS = BAGS_PER_W // CHUNK_BAGS


GATHER_SUB = 80
N_SUB = CHUNK_IDX // GATHER_SUB


def _ebag_kernel(x_hbm, table_hbm, out_hbm, idx_v, rows_v, out_v, sem):
    wid = lax.axis_index("s") * NC + lax.axis_index("c")

    def chunk_body(c, carry):
        idx_base = wid * (BAGS_PER_W * HIST) + c * CHUNK_IDX
        row_base = wid * BAGS_PER_W + c * CHUNK_BAGS

        pltpu.sync_copy(x_hbm.at[pl.ds(idx_base, CHUNK_IDX)], idx_v)

        def remap_body(k, carry2):
            sl = pl.ds(k * 16, 16)
            i = idx_v[sl]
            v = jnp.left_shift(jnp.bitwise_and(i, SPLIT - 1), 2)
            idx_v[sl] = jnp.bitwise_or(v, jnp.right_shift(i, 18))
            return carry2

        lax.fori_loop(0, CHUNK_IDX // 16, remap_body, 0, unroll=False)

        copies = []
        for j in range(N_SUB):
            sl = pl.ds(j * GATHER_SUB, GATHER_SUB)
            copies.append(
                pltpu.async_copy(table_hbm.at[idx_v.at[sl]], rows_v.at[sl], sem)
            )
        for cp in copies:
            cp.wait()

        def bag_body(r, carry2):
            base = r * HIST
            a = [jnp.zeros((16,), jnp.float32) for _ in range(8)]
            for j in range(HIST):
                p = (j % 4) * 2
                a[p] = a[p] + rows_v[base + j, pl.ds(0, 16)]
                a[p + 1] = a[p + 1] + rows_v[base + j, pl.ds(16, 16)]
            s0 = (a[0] + a[2]) + (a[4] + a[6])
            s1 = (a[1] + a[3]) + (a[5] + a[7])
            scale = jnp.float32(1.0 / HIST)
            out_v[r, pl.ds(0, 16)] = s0 * scale
            out_v[r, pl.ds(16, 16)] = s1 * scale
            return carry2

        lax.fori_loop(0, CHUNK_BAGS, bag_body, 0, unroll=False)

        pltpu.sync_copy(out_v, out_hbm.at[pl.ds(row_base, CHUNK_BAGS)])
        return carry

    lax.fori_loop(0, N_CHUNKS, chunk_body, 0, unroll=False)


@jax.jit
def kernel(x, table):
    table_rm = _table_to_split_rows(table)
    mesh = plsc.VectorSubcoreMesh(core_axis_name="c", subcore_axis_name="s")
    run = functools.partial(
        pl.kernel,
        mesh=mesh,
        out_type=jax.ShapeDtypeStruct((BATCH, DIM), jnp.float32),
        scratch_types=[
            pltpu.VMEM((CHUNK_IDX,), jnp.int32),
            pltpu.VMEM((CHUNK_IDX, DIM), jnp.float32),
            pltpu.VMEM((CHUNK_BAGS, DIM), jnp.float32),
            pltpu.SemaphoreType.DMA,
        ],
        compiler_params=pltpu.CompilerParams(use_tc_tiling_on_sc=False),
    )(_ebag_kernel)
    return run(x.reshape(-1), table_rm)


# speedup vs baseline: 4.1292x; 4.1292x over previous
import functools

import jax
import jax.numpy as jnp
from jax import lax
from jax.experimental import pallas as pl
from jax.experimental.pallas import tpu as pltpu
from jax.experimental.pallas import tpu_sc as plsc

BATCH = 16384
HIST = 50
DIM = 32
NUM_EMB = 1000000

SPLIT = 1 << 18
NSPLIT = 4
VROWS = SPLIT * NSPLIT
LAST_ALIGNED = NUM_EMB - (NUM_EMB % 128)
LAST_FULL = LAST_ALIGNED - 128

_info = plsc.get_sparse_core_info()
NC, NS = _info.num_cores, _info.num_subcores
NW = NC * NS

GROUPS = SPLIT // 128
GROUPS_PER_W = GROUPS // NW


def _fire_in_dmas(tt_hbm, in_v, isem, k, wid, buf):
    g = wid * GROUPS_PER_W + k
    q0 = g * 128
    for s in range(NSPLIT):
        c0 = jnp.minimum(s * SPLIT + q0, LAST_FULL)
        c0 = pl.multiple_of(c0, 128)
        pltpu.async_copy(tt_hbm.at[:, pl.ds(c0, 128)], in_v.at[buf, s], isem)


def _relayout_kernel(tt_hbm, out_hbm, in_v, in64_v, out_v, isem, osem):
    wid = lax.axis_index("s") * NC + lax.axis_index("c")
    iota = lax.iota(jnp.int32, 16)

    _fire_in_dmas(tt_hbm, in_v, isem, 0, wid, 0)

    def group_body(k, carry):
        b = k & 1
        g = wid * GROUPS_PER_W + k
        q0 = g * 128
        is_boundary = (3 * SPLIT + q0) == LAST_ALIGNED

        @pl.when(is_boundary)
        def _():
            pltpu.async_copy(
                tt_hbm.at[:, pl.ds(LAST_ALIGNED, NUM_EMB - LAST_ALIGNED)],
                in64_v,
                isem,
            ).wait()

        for s in range(NSPLIT):
            pltpu.make_async_copy(
                tt_hbm.at[:, pl.ds(0, 128)], in_v.at[b, s], isem
            ).wait()

        @pl.when(k + 1 < GROUPS_PER_W)
        def _():
            _fire_in_dmas(tt_hbm, in_v, isem, k + 1, wid, 1 - b)

        @pl.when(k >= 2)
        def _():
            pltpu.make_async_copy(
                out_v.at[b], out_hbm.at[pl.ds(0, 16384)], osem
            ).wait()

        @functools.partial(plsc.parallel_loop, 0, 128, unroll=4)
        def _(c):
            cvec = jnp.full((16,), c, jnp.int32)
            base = c * 128
            for s in range(NSPLIT):
                for h in range(2):
                    v = plsc.load_gather(in_v.at[b, s], [iota + (16 * h), cvec])
                    out_v[b, pl.ds(base + 32 * s + 16 * h, 16)] = v

        @pl.when(is_boundary)
        def _():
            def fix_body(c, carry2):
                cvec = jnp.full((16,), c, jnp.int32)
                for h in range(2):
                    v = plsc.load_gather(in64_v, [iota + (16 * h), cvec])
                    out_v[b, pl.ds(c * 128 + 96 + 16 * h, 16)] = v
                return carry2

            lax.fori_loop(0, NUM_EMB - LAST_ALIGNED, fix_body, 0,
                          unroll=False)

        pltpu.async_copy(out_v.at[b], out_hbm.at[pl.ds(q0 * 128, 16384)], osem)
        return carry

    lax.fori_loop(0, GROUPS_PER_W, group_body, 0, unroll=False)

    for _ in range(2):
        pltpu.make_async_copy(
            out_v.at[0], out_hbm.at[pl.ds(0, 16384)], osem
        ).wait()


def _table_to_split_rows(table):
    table_t = jnp.swapaxes(table, 0, 1)
    mesh = plsc.VectorSubcoreMesh(core_axis_name="c", subcore_axis_name="s")
    run = functools.partial(
        pl.kernel,
        mesh=mesh,
        out_type=jax.ShapeDtypeStruct((VROWS * DIM,), jnp.float32),
        scratch_types=[
            pltpu.VMEM((2, NSPLIT, DIM, 128), jnp.float32),
            pltpu.VMEM((DIM, NUM_EMB - LAST_ALIGNED), jnp.float32),
            pltpu.VMEM((2, 16384), jnp.float32),
            pltpu.SemaphoreType.DMA,
            pltpu.SemaphoreType.DMA,
        ],
        compiler_params=pltpu.CompilerParams(
            use_tc_tiling_on_sc=True, needs_layout_passes=False
        ),
    )(_relayout_kernel)
    return run(table_t).reshape(VROWS, DIM)


BAGS_PER_W = BATCH // NW
CHUNK_BAGS = 32
CHUNK_IDX = CHUNK_BAGS * HIST
N_CHUNKS = BAGS_PER_W // CHUNK_BAGS


GATHER_SUB = 80
N_SUB = CHUNK_IDX // GATHER_SUB


def _ebag_kernel(x_hbm, table_hbm, out_hbm, idx_v, rows_v, out_v, sem):
    wid = lax.axis_index("s") * NC + lax.axis_index("c")

    def chunk_body(c, carry):
        idx_base = wid * (BAGS_PER_W * HIST) + c * CHUNK_IDX
        row_base = wid * BAGS_PER_W + c * CHUNK_BAGS

        pltpu.sync_copy(x_hbm.at[pl.ds(idx_base, CHUNK_IDX)], idx_v)

        def remap_body(k, carry2):
            sl = pl.ds(k * 16, 16)
            i = idx_v[sl]
            v = jnp.left_shift(jnp.bitwise_and(i, SPLIT - 1), 2)
            idx_v[sl] = jnp.bitwise_or(v, jnp.right_shift(i, 18))
            return carry2

        lax.fori_loop(0, CHUNK_IDX // 16, remap_body, 0, unroll=False)

        copies = []
        for j in range(N_SUB):
            sl = pl.ds(j * GATHER_SUB, GATHER_SUB)
            copies.append(
                pltpu.async_copy(table_hbm.at[idx_v.at[sl]], rows_v.at[sl], sem)
            )
        for cp in copies:
            cp.wait()

        def bag_body(r, carry2):
            base = r * HIST
            a = [jnp.zeros((16,), jnp.float32) for _ in range(8)]
            for j in range(HIST):
                p = (j % 4) * 2
                a[p] = a[p] + rows_v[base + j, pl.ds(0, 16)]
                a[p + 1] = a[p + 1] + rows_v[base + j, pl.ds(16, 16)]
            s0 = (a[0] + a[2]) + (a[4] + a[6])
            s1 = (a[1] + a[3]) + (a[5] + a[7])
            scale = jnp.float32(1.0 / HIST)
            out_v[r, pl.ds(0, 16)] = s0 * scale
            out_v[r, pl.ds(16, 16)] = s1 * scale
            return carry2

        lax.fori_loop(0, CHUNK_BAGS, bag_body, 0, unroll=False)

        pltpu.sync_copy(out_v, out_hbm.at[pl.ds(row_base, CHUNK_BAGS)])
        return carry

    lax.fori_loop(0, N_CHUNKS, chunk_body, 0, unroll=False)


@jax.jit
def kernel(x, table):
    table_rm = _table_to_split_rows(table)
    mesh = plsc.VectorSubcoreMesh(core_axis_name="c", subcore_axis_name="s")
    run = functools.partial(
        pl.kernel,
        mesh=mesh,
        out_type=jax.ShapeDtypeStruct((BATCH, DIM), jnp.float32),
        scratch_types=[
            pltpu.VMEM((CHUNK_IDX,), jnp.int32),
            pltpu.VMEM((CHUNK_IDX, DIM), jnp.float32),
            pltpu.VMEM((CHUNK_BAGS, DIM), jnp.float32),
            pltpu.SemaphoreType.DMA,
        ],
        compiler_params=pltpu.CompilerParams(use_tc_tiling_on_sc=False),
    )(_ebag_kernel)
    return run(x.reshape(-1), table_rm)


# speedup vs baseline: 5.2413x; 1.2693x over previous
import functools

import jax
import jax.numpy as jnp
from jax import lax
from jax.experimental import pallas as pl
from jax.experimental.pallas import tpu as pltpu
from jax.experimental.pallas import tpu_sc as plsc

BATCH = 16384
HIST = 50
DIM = 32
NUM_EMB = 1000000

SPLIT = 1 << 18
NSPLIT = 4
VROWS = SPLIT * NSPLIT
LAST_ALIGNED = NUM_EMB - (NUM_EMB % 128)
LAST_FULL = LAST_ALIGNED - 128

_info = plsc.get_sparse_core_info()
NC, NS = _info.num_cores, _info.num_subcores
NW = NC * NS

GROUPS = SPLIT // 128
GROUPS_PER_W = GROUPS // NW


def _fire_in_dmas(tt_hbm, in_v, isem, k, wid, buf):
    g = wid * GROUPS_PER_W + k
    q0 = g * 128
    for s in range(NSPLIT):
        c0 = jnp.minimum(s * SPLIT + q0, LAST_FULL)
        c0 = pl.multiple_of(c0, 128)
        pltpu.async_copy(tt_hbm.at[:, pl.ds(c0, 128)], in_v.at[buf, s], isem)


def _relayout_kernel(tt_hbm, out_hbm, in_v, in64_v, out_v, isem, osem):
    wid = lax.axis_index("s") * NC + lax.axis_index("c")
    iota = lax.iota(jnp.int32, 16)

    _fire_in_dmas(tt_hbm, in_v, isem, 0, wid, 0)

    def group_body(k, carry):
        b = k & 1
        g = wid * GROUPS_PER_W + k
        q0 = g * 128
        is_boundary = (3 * SPLIT + q0) == LAST_ALIGNED

        @pl.when(is_boundary)
        def _():
            pltpu.async_copy(
                tt_hbm.at[:, pl.ds(LAST_ALIGNED, NUM_EMB - LAST_ALIGNED)],
                in64_v,
                isem,
            ).wait()

        for s in range(NSPLIT):
            pltpu.make_async_copy(
                tt_hbm.at[:, pl.ds(0, 128)], in_v.at[b, s], isem
            ).wait()

        @pl.when(k + 1 < GROUPS_PER_W)
        def _():
            _fire_in_dmas(tt_hbm, in_v, isem, k + 1, wid, 1 - b)

        @pl.when(k >= 2)
        def _():
            pltpu.make_async_copy(
                out_v.at[b], out_hbm.at[pl.ds(0, 16384)], osem
            ).wait()

        @functools.partial(plsc.parallel_loop, 0, 128, unroll=4)
        def _(c):
            cvec = jnp.full((16,), c, jnp.int32)
            base = c * 128
            for s in range(NSPLIT):
                for h in range(2):
                    v = plsc.load_gather(in_v.at[b, s], [iota + (16 * h), cvec])
                    out_v[b, pl.ds(base + 32 * s + 16 * h, 16)] = v

        @pl.when(is_boundary)
        def _():
            def fix_body(c, carry2):
                cvec = jnp.full((16,), c, jnp.int32)
                for h in range(2):
                    v = plsc.load_gather(in64_v, [iota + (16 * h), cvec])
                    out_v[b, pl.ds(c * 128 + 96 + 16 * h, 16)] = v
                return carry2

            lax.fori_loop(0, NUM_EMB - LAST_ALIGNED, fix_body, 0,
                          unroll=False)

        pltpu.async_copy(out_v.at[b], out_hbm.at[pl.ds(q0 * 128, 16384)], osem)
        return carry

    lax.fori_loop(0, GROUPS_PER_W, group_body, 0, unroll=False)

    for _ in range(2):
        pltpu.make_async_copy(
            out_v.at[0], out_hbm.at[pl.ds(0, 16384)], osem
        ).wait()


def _table_to_split_rows(table):
    table_t = jnp.swapaxes(table, 0, 1)
    mesh = plsc.VectorSubcoreMesh(core_axis_name="c", subcore_axis_name="s")
    run = functools.partial(
        pl.kernel,
        mesh=mesh,
        out_type=jax.ShapeDtypeStruct((VROWS * DIM,), jnp.float32),
        scratch_types=[
            pltpu.VMEM((2, NSPLIT, DIM, 128), jnp.float32),
            pltpu.VMEM((DIM, NUM_EMB - LAST_ALIGNED), jnp.float32),
            pltpu.VMEM((2, 16384), jnp.float32),
            pltpu.SemaphoreType.DMA,
            pltpu.SemaphoreType.DMA,
        ],
        compiler_params=pltpu.CompilerParams(
            use_tc_tiling_on_sc=True, needs_layout_passes=False
        ),
    )(_relayout_kernel)
    return run(table_t).reshape(VROWS, DIM)


BAGS_PER_W = BATCH // NW
CHUNK_BAGS = 32
CHUNK_IDX = CHUNK_BAGS * HIST
N_CHUNKS = BAGS_PER_W // CHUNK_BAGS


GATHER_SUB = 80
N_SUB = CHUNK_IDX // GATHER_SUB


def _ebag_kernel(x_hbm, table_hbm, out_hbm, idx_v, rows_v, out_v,
                 sem0, sem1):
    wid = lax.axis_index("s") * NC + lax.axis_index("c")
    sems = [sem0, sem1]

    def stage_chunk(c, b):
        idx_base = wid * (BAGS_PER_W * HIST) + c * CHUNK_IDX
        pltpu.sync_copy(x_hbm.at[pl.ds(idx_base, CHUNK_IDX)], idx_v.at[b])

        @functools.partial(plsc.parallel_loop, 0, CHUNK_IDX // 16, unroll=4)
        def _(k):
            sl = pl.ds(k * 16, 16)
            i = idx_v[b, sl]
            v = jnp.left_shift(jnp.bitwise_and(i, SPLIT - 1), 2)
            idx_v[b, sl] = jnp.bitwise_or(v, jnp.right_shift(i, 18))

        for j in range(N_SUB):
            sl = pl.ds(j * GATHER_SUB, GATHER_SUB)
            pltpu.async_copy(
                table_hbm.at[idx_v.at[b, sl]], rows_v.at[b, sl], sems[b]
            )

    stage_chunk(0, 0)

    def super_body(k, carry):
        for b in range(2):
            c = 2 * k + b

            @pl.when(c + 1 < N_CHUNKS)
            def _():
                stage_chunk(c + 1, 1 - b)

            for j in range(N_SUB):
                sl = pl.ds(j * GATHER_SUB, GATHER_SUB)
                pltpu.make_async_copy(
                    table_hbm.at[idx_v.at[b, sl]], rows_v.at[b, sl], sems[b]
                ).wait()

            @functools.partial(plsc.parallel_loop, 0, CHUNK_BAGS, unroll=2)
            def _(r):
                base = r * HIST
                a = [jnp.zeros((16,), jnp.float32) for _ in range(8)]
                for j in range(HIST):
                    p = (j % 4) * 2
                    a[p] = a[p] + rows_v[b, base + j, pl.ds(0, 16)]
                    a[p + 1] = a[p + 1] + rows_v[b, base + j, pl.ds(16, 16)]
                s0 = (a[0] + a[2]) + (a[4] + a[6])
                s1 = (a[1] + a[3]) + (a[5] + a[7])
                scale = jnp.float32(1.0 / HIST)
                out_v[r, pl.ds(0, 16)] = s0 * scale
                out_v[r, pl.ds(16, 16)] = s1 * scale

            row_base = wid * BAGS_PER_W + c * CHUNK_BAGS
            pltpu.sync_copy(out_v, out_hbm.at[pl.ds(row_base, CHUNK_BAGS)])
        return carry

    lax.fori_loop(0, N_CHUNKS // 2, super_body, 0, unroll=False)


@jax.jit
def kernel(x, table):
    table_rm = _table_to_split_rows(table)
    mesh = plsc.VectorSubcoreMesh(core_axis_name="c", subcore_axis_name="s")
    run = functools.partial(
        pl.kernel,
        mesh=mesh,
        out_type=jax.ShapeDtypeStruct((BATCH, DIM), jnp.float32),
        scratch_types=[
            pltpu.VMEM((2, CHUNK_IDX), jnp.int32),
            pltpu.VMEM((2, CHUNK_IDX, DIM), jnp.float32),
            pltpu.VMEM((CHUNK_BAGS, DIM), jnp.float32),
            pltpu.SemaphoreType.DMA,
            pltpu.SemaphoreType.DMA,
        ],
        compiler_params=pltpu.CompilerParams(use_tc_tiling_on_sc=False),
    )(_ebag_kernel)
    return run(x.reshape(-1), table_rm)
